# baseline (speedup 1.0000x reference)
import jax
import jax.numpy as jnp
from jax import lax
from jax.experimental import pallas as pl
from jax.experimental.pallas import tpu as pltpu
from jax.experimental.pallas import tpu_sc as plsc

N_NODES = 10000
D_FEAT = 128
N_EDGES = 320000

NC = 2
NS = 16
NW = NC * NS
E_PER_W = N_EDGES // NW
CHUNK = 80
E_PER_W_PAD = 10240
N_CHUNKS = E_PER_W_PAD // CHUNK
BLK = 16
N_BLKS = N_CHUNKS // BLK
N_PAD = 10240
ROWS_PER_TILE = N_PAD // NS
DUMMY_ROW = N_NODES


def _sc_partials_kernel(x_hbm, src_hbm, dst_hbm, zeros_hbm, out_hbm,
                        src_idx, dblk0, dblk1, rows0, rows1, acc,
                        gsem0, gsem1, ssem0, ssem1, dsem0, dsem1):
  cid = lax.axis_index("c")
  sid = lax.axis_index("s")
  wid = sid * NC + cid
  bufs = ((rows0, gsem0, ssem0), (rows1, gsem1, ssem1))
  dblks = ((dblk0, dsem0), (dblk1, dsem1))

  def gather_start(j, buf, gsem):
    pltpu.async_copy(x_hbm.at[src_idx.at[j]], buf, gsem)

  def gather_wait(buf, gsem):
    pltpu.make_async_copy(x_hbm.at[src_idx.at[0]], buf, gsem).wait()

  def scatter_start(dref, buf, ssem):
    pltpu.async_copy(buf, acc.at[dref], ssem, add=True)

  def scatter_wait(buf, ssem):
    pltpu.make_async_copy(buf, acc.at[dblk0.at[0]], ssem).wait()

  pltpu.sync_copy(zeros_hbm, acc.at[pl.ds(sid * ROWS_PER_TILE, ROWS_PER_TILE)])
  pltpu.sync_copy(src_hbm.at[wid], src_idx)
  pltpu.sync_copy(dst_hbm.at[wid, pl.ds(0, BLK)], dblk0)
  plsc.subcore_barrier()

  for k in range(N_BLKS):
    dbuf, dsem = dblks[k % 2]
    if k + 1 < N_BLKS:
      nbuf, nsem = dblks[(k + 1) % 2]
      pltpu.async_copy(dst_hbm.at[wid, pl.ds((k + 1) * BLK, BLK)], nbuf, nsem)
    if k > 0:
      pltpu.make_async_copy(dst_hbm.at[wid, pl.ds(0, BLK)], dbuf, dsem).wait()

    def chunk_step(j, carry, k=k, dbuf=dbuf):
      pltpu.async_copy(x_hbm.at[src_idx.at[k * BLK + j]], rows0, gsem0).wait()
      pltpu.sync_copy(rows0, acc.at[dbuf.at[j]], add=True)
      return carry

    lax.fori_loop(0, BLK, chunk_step, 0)

  plsc.subcore_barrier()
  pltpu.sync_copy(acc.at[pl.ds(sid * ROWS_PER_TILE, ROWS_PER_TILE)],
                  out_hbm.at[cid, pl.ds(sid * ROWS_PER_TILE, ROWS_PER_TILE)])


def _combine_kernel(eps_ref, x_ref, p_ref, o_ref):
  scale = 1.0 + eps_ref[0]
  o_ref[...] = x_ref[...] * scale + p_ref[0] + p_ref[1]


@jax.jit
def kernel(x, edge_index, eps):
  n_pad_edges = E_PER_W_PAD - E_PER_W
  src = edge_index[0].astype(jnp.int32).reshape(NW, E_PER_W)
  dst = edge_index[1].astype(jnp.int32).reshape(NW, E_PER_W)
  src = jnp.concatenate(
      [src, jnp.zeros((NW, n_pad_edges), jnp.int32)], axis=1)
  pad_dst = DUMMY_ROW + jnp.arange(n_pad_edges, dtype=jnp.int32) % (N_PAD - N_NODES)
  dst = jnp.concatenate(
      [dst, jnp.broadcast_to(pad_dst, (NW, n_pad_edges))], axis=1)
  src = src.reshape(NW, N_CHUNKS, CHUNK)
  dst = dst.reshape(NW, N_CHUNKS, CHUNK)
  zeros = jnp.zeros((ROWS_PER_TILE, D_FEAT), dtype=jnp.float32)

  mesh = plsc.VectorSubcoreMesh(core_axis_name="c", subcore_axis_name="s")
  partials = pl.kernel(
      _sc_partials_kernel,
      out_type=jax.ShapeDtypeStruct((NC, N_PAD, D_FEAT), jnp.float32),
      mesh=mesh,
      scratch_types=[
          pltpu.VMEM((N_CHUNKS, CHUNK), jnp.int32),
          pltpu.VMEM((BLK, CHUNK), jnp.int32),
          pltpu.VMEM((BLK, CHUNK), jnp.int32),
          pltpu.VMEM((CHUNK, D_FEAT), jnp.float32),
          pltpu.VMEM((CHUNK, D_FEAT), jnp.float32),
          pltpu.VMEM_SHARED((N_PAD, D_FEAT), jnp.float32),
          pltpu.SemaphoreType.DMA,
          pltpu.SemaphoreType.DMA,
          pltpu.SemaphoreType.DMA,
          pltpu.SemaphoreType.DMA,
          pltpu.SemaphoreType.DMA,
          pltpu.SemaphoreType.DMA,
      ],
  )(x, src, dst, zeros)

  rows_blk = 1000
  grid = N_NODES // rows_blk
  out = pl.pallas_call(
      _combine_kernel,
      out_shape=jax.ShapeDtypeStruct((N_NODES, D_FEAT), jnp.float32),
      grid=(grid,),
      in_specs=[
          pl.BlockSpec(memory_space=pltpu.SMEM),
          pl.BlockSpec((rows_blk, D_FEAT), lambda i: (i, 0)),
          pl.BlockSpec((NC, rows_blk, D_FEAT), lambda i: (0, i, 0)),
      ],
      out_specs=pl.BlockSpec((rows_blk, D_FEAT), lambda i: (i, 0)),
  )(eps, x, partials)
  return out


# speedup vs baseline: 2.0749x; 2.0749x over previous
import jax
import jax.numpy as jnp
from jax import lax
from jax.experimental import pallas as pl
from jax.experimental.pallas import tpu as pltpu
from jax.experimental.pallas import tpu_sc as plsc

N_NODES = 10000
D_FEAT = 128
N_EDGES = 320000

NC = 2
NS = 16
NW = NC * NS
E_PER_W = N_EDGES // NW
CHUNK = 80
E_PER_W_PAD = 10240
N_CHUNKS = E_PER_W_PAD // CHUNK
BLK = 16
N_BLKS = N_CHUNKS // BLK
N_PAD = 10240
ROWS_PER_TILE = N_PAD // NS
DUMMY_ROW = N_NODES


def _sc_partials_kernel(x_hbm, src_hbm, dst_hbm, zeros_hbm, out_hbm,
                        src_idx, dblk0, dblk1, rows0, rows1, acc,
                        gsem0, gsem1, ssem0, ssem1, dsem0, dsem1):
  cid = lax.axis_index("c")
  sid = lax.axis_index("s")
  wid = sid * NC + cid
  bufs = ((rows0, gsem0, ssem0), (rows1, gsem1, ssem1))
  dblks = ((dblk0, dsem0), (dblk1, dsem1))

  def gather_start(j, buf, gsem):
    pltpu.async_copy(x_hbm.at[src_idx.at[j]], buf, gsem)

  def gather_wait(buf, gsem):
    pltpu.make_async_copy(x_hbm.at[src_idx.at[0]], buf, gsem).wait()

  def scatter_start(dref, buf, ssem):
    pltpu.async_copy(buf, acc.at[dref], ssem, add=True)

  def scatter_wait(buf, ssem):
    pltpu.make_async_copy(buf, acc.at[dblk0.at[0]], ssem).wait()

  pltpu.sync_copy(zeros_hbm, acc.at[pl.ds(sid * ROWS_PER_TILE, ROWS_PER_TILE)])
  pltpu.sync_copy(src_hbm.at[wid], src_idx)
  pltpu.sync_copy(dst_hbm.at[wid, pl.ds(0, BLK)], dblk0)
  plsc.subcore_barrier()

  for k in range(N_BLKS):
    dbuf, dsem = dblks[k % 2]
    if k + 1 < N_BLKS:
      nbuf, nsem = dblks[(k + 1) % 2]
      pltpu.async_copy(dst_hbm.at[wid, pl.ds((k + 1) * BLK, BLK)], nbuf, nsem)
    if k > 0:
      pltpu.make_async_copy(dst_hbm.at[wid, pl.ds(0, BLK)], dbuf, dsem).wait()

    def chunk_step(j, carry, k=k, dbuf=dbuf):
      pltpu.async_copy(x_hbm.at[src_idx.at[k * BLK + j]], rows0, gsem0).wait()
      pltpu.sync_copy(rows0, acc.at[dbuf.at[j]], add=True)
      return carry

    lax.fori_loop(0, BLK, chunk_step, 0)

  plsc.subcore_barrier()
  pltpu.sync_copy(acc.at[pl.ds(sid * ROWS_PER_TILE, ROWS_PER_TILE)],
                  out_hbm.at[cid, pl.ds(sid * ROWS_PER_TILE, ROWS_PER_TILE)])


def _combine_kernel(eps_ref, x_ref, p_ref, o_ref):
  scale = 1.0 + eps_ref[0]
  o_ref[...] = x_ref[...] * scale + p_ref[0] + p_ref[1]


@jax.jit
def kernel(x, edge_index, eps):
  n_pad_edges = E_PER_W_PAD - E_PER_W
  src = edge_index[0].astype(jnp.int32).reshape(NW, E_PER_W)
  dst = edge_index[1].astype(jnp.int32).reshape(NW, E_PER_W)
  pad_src = jnp.arange(n_pad_edges, dtype=jnp.int32) * 37 % N_NODES
  src = jnp.concatenate(
      [src, jnp.broadcast_to(pad_src, (NW, n_pad_edges))], axis=1)
  pad_dst = DUMMY_ROW + jnp.arange(n_pad_edges, dtype=jnp.int32) % (N_PAD - N_NODES)
  dst = jnp.concatenate(
      [dst, jnp.broadcast_to(pad_dst, (NW, n_pad_edges))], axis=1)
  src = src.reshape(NW, N_CHUNKS, CHUNK)
  dst = dst.reshape(NW, N_CHUNKS, CHUNK)
  zeros = jnp.zeros((ROWS_PER_TILE, D_FEAT), dtype=jnp.float32)

  mesh = plsc.VectorSubcoreMesh(core_axis_name="c", subcore_axis_name="s")
  partials = pl.kernel(
      _sc_partials_kernel,
      out_type=jax.ShapeDtypeStruct((NC, N_PAD, D_FEAT), jnp.float32),
      mesh=mesh,
      scratch_types=[
          pltpu.VMEM((N_CHUNKS, CHUNK), jnp.int32),
          pltpu.VMEM((BLK, CHUNK), jnp.int32),
          pltpu.VMEM((BLK, CHUNK), jnp.int32),
          pltpu.VMEM((CHUNK, D_FEAT), jnp.float32),
          pltpu.VMEM((CHUNK, D_FEAT), jnp.float32),
          pltpu.VMEM_SHARED((N_PAD, D_FEAT), jnp.float32),
          pltpu.SemaphoreType.DMA,
          pltpu.SemaphoreType.DMA,
          pltpu.SemaphoreType.DMA,
          pltpu.SemaphoreType.DMA,
          pltpu.SemaphoreType.DMA,
          pltpu.SemaphoreType.DMA,
      ],
  )(x, src, dst, zeros)

  rows_blk = 1000
  grid = N_NODES // rows_blk
  out = pl.pallas_call(
      _combine_kernel,
      out_shape=jax.ShapeDtypeStruct((N_NODES, D_FEAT), jnp.float32),
      grid=(grid,),
      in_specs=[
          pl.BlockSpec(memory_space=pltpu.SMEM),
          pl.BlockSpec((rows_blk, D_FEAT), lambda i: (i, 0)),
          pl.BlockSpec((NC, rows_blk, D_FEAT), lambda i: (0, i, 0)),
      ],
      out_specs=pl.BlockSpec((rows_blk, D_FEAT), lambda i: (i, 0)),
  )(eps, x, partials)
  return out


# speedup vs baseline: 2.5779x; 1.2425x over previous
import jax
import jax.numpy as jnp
from jax import lax
from jax.experimental import pallas as pl
from jax.experimental.pallas import tpu as pltpu
from jax.experimental.pallas import tpu_sc as plsc

N_NODES = 10000
D_FEAT = 128
N_EDGES = 320000

NC = 2
NS = 16
NW = NC * NS
E_PER_W = N_EDGES // NW
CHUNK = 80
E_PER_W_PAD = 10240
N_CHUNKS = E_PER_W_PAD // CHUNK
BLK = 16
N_BLKS = N_CHUNKS // BLK
N_PAD = 10240
ROWS_PER_TILE = N_PAD // NS
DUMMY_ROW = N_NODES


def _sc_partials_kernel(x_hbm, src_hbm, dst_hbm, zeros_hbm, out_hbm,
                        src_idx, dblk0, dblk1, rows0, rows1, acc,
                        gsem0, gsem1, ssem0, ssem1, dsem0, dsem1):
  cid = lax.axis_index("c")
  sid = lax.axis_index("s")
  wid = sid * NC + cid
  bufs = ((rows0, gsem0, ssem0), (rows1, gsem1, ssem1))
  dblks = ((dblk0, dsem0), (dblk1, dsem1))

  def gather_start(j, buf, gsem):
    pltpu.async_copy(x_hbm.at[src_idx.at[j]], buf, gsem)

  def gather_wait(buf, gsem):
    pltpu.make_async_copy(x_hbm.at[src_idx.at[0]], buf, gsem).wait()

  def scatter_start(dref, buf, ssem):
    pltpu.async_copy(buf, acc.at[dref], ssem, add=True)

  def scatter_wait(buf, ssem):
    pltpu.make_async_copy(buf, acc.at[dblk0.at[0]], ssem).wait()

  pltpu.sync_copy(zeros_hbm, acc.at[pl.ds(sid * ROWS_PER_TILE, ROWS_PER_TILE)])
  pltpu.sync_copy(src_hbm.at[wid], src_idx)
  pltpu.sync_copy(dst_hbm.at[wid, pl.ds(0, BLK)], dblk0)
  plsc.subcore_barrier()

  gather_start(0, rows0, gsem0)
  for k in range(N_BLKS):
    dbuf, dsem = dblks[k % 2]
    if k + 1 < N_BLKS:
      nbuf, nsem = dblks[(k + 1) % 2]
      pltpu.async_copy(dst_hbm.at[wid, pl.ds((k + 1) * BLK, BLK)], nbuf, nsem)
    if k > 0:
      pltpu.make_async_copy(dst_hbm.at[wid, pl.ds(0, BLK)], dbuf, dsem).wait()

    def pair_step(j2, carry, k=k, dbuf=dbuf):
      base = k * BLK + 2 * j2
      for b, (buf, gsem, ssem) in enumerate(bufs):
        nbuf_, ngsem, _ = bufs[1 - b]
        gather_wait(buf, gsem)
        scatter_start(dbuf.at[2 * j2 + b], buf, ssem)

        @pl.when(base + b + 1 < N_CHUNKS)
        def _next(nbuf_=nbuf_, ngsem=ngsem):
          gather_start(base + b + 1, nbuf_, ngsem)

        scatter_wait(buf, ssem)
      return carry

    lax.fori_loop(0, BLK // 2, pair_step, 0)

  plsc.subcore_barrier()
  pltpu.sync_copy(acc.at[pl.ds(sid * ROWS_PER_TILE, ROWS_PER_TILE)],
                  out_hbm.at[cid, pl.ds(sid * ROWS_PER_TILE, ROWS_PER_TILE)])


def _combine_kernel(eps_ref, x_ref, p_ref, o_ref):
  scale = 1.0 + eps_ref[0]
  o_ref[...] = x_ref[...] * scale + p_ref[0] + p_ref[1]


@jax.jit
def kernel(x, edge_index, eps):
  n_pad_edges = E_PER_W_PAD - E_PER_W
  src = edge_index[0].astype(jnp.int32).reshape(NW, E_PER_W)
  dst = edge_index[1].astype(jnp.int32).reshape(NW, E_PER_W)
  pad_src = jnp.arange(n_pad_edges, dtype=jnp.int32) * 37 % N_NODES
  src = jnp.concatenate(
      [src, jnp.broadcast_to(pad_src, (NW, n_pad_edges))], axis=1)
  pad_dst = DUMMY_ROW + jnp.arange(n_pad_edges, dtype=jnp.int32) % (N_PAD - N_NODES)
  dst = jnp.concatenate(
      [dst, jnp.broadcast_to(pad_dst, (NW, n_pad_edges))], axis=1)
  src = src.reshape(NW, N_CHUNKS, CHUNK)
  dst = dst.reshape(NW, N_CHUNKS, CHUNK)
  zeros = jnp.zeros((ROWS_PER_TILE, D_FEAT), dtype=jnp.float32)

  mesh = plsc.VectorSubcoreMesh(core_axis_name="c", subcore_axis_name="s")
  partials = pl.kernel(
      _sc_partials_kernel,
      out_type=jax.ShapeDtypeStruct((NC, N_PAD, D_FEAT), jnp.float32),
      mesh=mesh,
      scratch_types=[
          pltpu.VMEM((N_CHUNKS, CHUNK), jnp.int32),
          pltpu.VMEM((BLK, CHUNK), jnp.int32),
          pltpu.VMEM((BLK, CHUNK), jnp.int32),
          pltpu.VMEM((CHUNK, D_FEAT), jnp.float32),
          pltpu.VMEM((CHUNK, D_FEAT), jnp.float32),
          pltpu.VMEM_SHARED((N_PAD, D_FEAT), jnp.float32),
          pltpu.SemaphoreType.DMA,
          pltpu.SemaphoreType.DMA,
          pltpu.SemaphoreType.DMA,
          pltpu.SemaphoreType.DMA,
          pltpu.SemaphoreType.DMA,
          pltpu.SemaphoreType.DMA,
      ],
  )(x, src, dst, zeros)

  rows_blk = 1000
  grid = N_NODES // rows_blk
  out = pl.pallas_call(
      _combine_kernel,
      out_shape=jax.ShapeDtypeStruct((N_NODES, D_FEAT), jnp.float32),
      grid=(grid,),
      in_specs=[
          pl.BlockSpec(memory_space=pltpu.SMEM),
          pl.BlockSpec((rows_blk, D_FEAT), lambda i: (i, 0)),
          pl.BlockSpec((NC, rows_blk, D_FEAT), lambda i: (0, i, 0)),
      ],
      out_specs=pl.BlockSpec((rows_blk, D_FEAT), lambda i: (i, 0)),
  )(eps, x, partials)
  return out


# speedup vs baseline: 2.9920x; 1.1606x over previous
import jax
import jax.numpy as jnp
from jax import lax
from jax.experimental import pallas as pl
from jax.experimental.pallas import tpu as pltpu
from jax.experimental.pallas import tpu_sc as plsc

N_NODES = 10000
D_FEAT = 128
N_EDGES = 320000

NC = 2
NS = 16
NW = NC * NS
E_PER_W = N_EDGES // NW
CHUNK = 128
E_PER_W_PAD = 10240
N_CHUNKS = E_PER_W_PAD // CHUNK
BLK = 16
N_BLKS = N_CHUNKS // BLK
N_PAD = 10240
ROWS_PER_TILE = N_PAD // NS
DUMMY_ROW = N_NODES


def _sc_partials_kernel(x_hbm, src_hbm, dst_hbm, zeros_hbm, out_hbm,
                        src_idx, dblk0, dblk1, rows0, rows1, acc,
                        gsem0, gsem1, ssem0, ssem1, dsem0, dsem1):
  cid = lax.axis_index("c")
  sid = lax.axis_index("s")
  wid = sid * NC + cid
  bufs = ((rows0, gsem0, ssem0), (rows1, gsem1, ssem1))
  dblks = ((dblk0, dsem0), (dblk1, dsem1))

  def gather_start(j, buf, gsem):
    pltpu.async_copy(x_hbm.at[src_idx.at[j]], buf, gsem)

  def gather_wait(buf, gsem):
    pltpu.make_async_copy(x_hbm.at[src_idx.at[0]], buf, gsem).wait()

  def scatter_start(dref, buf, ssem):
    pltpu.async_copy(buf, acc.at[dref], ssem, add=True)

  def scatter_wait(buf, ssem):
    pltpu.make_async_copy(buf, acc.at[dblk0.at[0]], ssem).wait()

  pltpu.sync_copy(zeros_hbm, acc.at[pl.ds(sid * ROWS_PER_TILE, ROWS_PER_TILE)])
  pltpu.sync_copy(src_hbm.at[wid], src_idx)
  pltpu.sync_copy(dst_hbm.at[wid, pl.ds(0, BLK)], dblk0)
  plsc.subcore_barrier()

  gather_start(0, rows0, gsem0)
  for k in range(N_BLKS):
    dbuf, dsem = dblks[k % 2]
    if k + 1 < N_BLKS:
      nbuf, nsem = dblks[(k + 1) % 2]
      pltpu.async_copy(dst_hbm.at[wid, pl.ds((k + 1) * BLK, BLK)], nbuf, nsem)
    if k > 0:
      pltpu.make_async_copy(dst_hbm.at[wid, pl.ds(0, BLK)], dbuf, dsem).wait()

    def pair_step(j2, carry, k=k, dbuf=dbuf):
      base = k * BLK + 2 * j2
      for b, (buf, gsem, ssem) in enumerate(bufs):
        nbuf_, ngsem, _ = bufs[1 - b]
        gather_wait(buf, gsem)
        scatter_start(dbuf.at[2 * j2 + b], buf, ssem)

        @pl.when(base + b + 1 < N_CHUNKS)
        def _next(nbuf_=nbuf_, ngsem=ngsem):
          gather_start(base + b + 1, nbuf_, ngsem)

        scatter_wait(buf, ssem)
      return carry

    lax.fori_loop(0, BLK // 2, pair_step, 0)

  plsc.subcore_barrier()
  pltpu.sync_copy(acc.at[pl.ds(sid * ROWS_PER_TILE, ROWS_PER_TILE)],
                  out_hbm.at[cid, pl.ds(sid * ROWS_PER_TILE, ROWS_PER_TILE)])


def _combine_kernel(eps_ref, x_ref, p_ref, o_ref):
  scale = 1.0 + eps_ref[0]
  o_ref[...] = x_ref[...] * scale + p_ref[0] + p_ref[1]


@jax.jit
def kernel(x, edge_index, eps):
  n_pad_edges = E_PER_W_PAD - E_PER_W
  src = edge_index[0].astype(jnp.int32).reshape(NW, E_PER_W)
  dst = edge_index[1].astype(jnp.int32).reshape(NW, E_PER_W)
  pad_src = jnp.arange(n_pad_edges, dtype=jnp.int32) * 37 % N_NODES
  src = jnp.concatenate(
      [src, jnp.broadcast_to(pad_src, (NW, n_pad_edges))], axis=1)
  pad_dst = DUMMY_ROW + jnp.arange(n_pad_edges, dtype=jnp.int32) % (N_PAD - N_NODES)
  dst = jnp.concatenate(
      [dst, jnp.broadcast_to(pad_dst, (NW, n_pad_edges))], axis=1)
  src = src.reshape(NW, N_CHUNKS, CHUNK)
  dst = dst.reshape(NW, N_CHUNKS, CHUNK)
  zeros = jnp.zeros((ROWS_PER_TILE, D_FEAT), dtype=jnp.float32)

  mesh = plsc.VectorSubcoreMesh(core_axis_name="c", subcore_axis_name="s")
  partials = pl.kernel(
      _sc_partials_kernel,
      out_type=jax.ShapeDtypeStruct((NC, N_PAD, D_FEAT), jnp.float32),
      mesh=mesh,
      scratch_types=[
          pltpu.VMEM((N_CHUNKS, CHUNK), jnp.int32),
          pltpu.VMEM((BLK, CHUNK), jnp.int32),
          pltpu.VMEM((BLK, CHUNK), jnp.int32),
          pltpu.VMEM((CHUNK, D_FEAT), jnp.float32),
          pltpu.VMEM((CHUNK, D_FEAT), jnp.float32),
          pltpu.VMEM_SHARED((N_PAD, D_FEAT), jnp.float32),
          pltpu.SemaphoreType.DMA,
          pltpu.SemaphoreType.DMA,
          pltpu.SemaphoreType.DMA,
          pltpu.SemaphoreType.DMA,
          pltpu.SemaphoreType.DMA,
          pltpu.SemaphoreType.DMA,
      ],
  )(x, src, dst, zeros)

  rows_blk = 1000
  grid = N_NODES // rows_blk
  out = pl.pallas_call(
      _combine_kernel,
      out_shape=jax.ShapeDtypeStruct((N_NODES, D_FEAT), jnp.float32),
      grid=(grid,),
      in_specs=[
          pl.BlockSpec(memory_space=pltpu.SMEM),
          pl.BlockSpec((rows_blk, D_FEAT), lambda i: (i, 0)),
          pl.BlockSpec((NC, rows_blk, D_FEAT), lambda i: (0, i, 0)),
      ],
      out_specs=pl.BlockSpec((rows_blk, D_FEAT), lambda i: (i, 0)),
  )(eps, x, partials)
  return out
